# initial kernel scaffold (unmeasured)
import jax
import jax.numpy as jnp
from jax import lax
from jax.experimental import pallas as pl
from jax.experimental.pallas import tpu as pltpu

N_DEV = 8
SCALE = 0.08838834764831843
BLK = 64
QT = 512


def _bcast_body(k_ref, v_ref, kout_ref, vout_ref, send_sems, recv_sems):
    my = lax.axis_index("i")
    right = lax.rem(my + 1, N_DEV)

    copy_k = pltpu.make_async_remote_copy(
        src_ref=kout_ref,
        dst_ref=kout_ref,
        send_sem=send_sems.at[0],
        recv_sem=recv_sems.at[0],
        device_id=(right,),
        device_id_type=pl.DeviceIdType.MESH,
    )
    copy_v = pltpu.make_async_remote_copy(
        src_ref=vout_ref,
        dst_ref=vout_ref,
        send_sem=send_sems.at[1],
        recv_sem=recv_sems.at[1],
        device_id=(right,),
        device_id_type=pl.DeviceIdType.MESH,
    )

    @pl.when(my == 0)
    def _():
        kout_ref[...] = k_ref[...]
        vout_ref[...] = v_ref[...]

    @pl.when(my > 0)
    def _():
        copy_k.wait_recv()
        copy_v.wait_recv()

    @pl.when(my < N_DEV - 1)
    def _():
        copy_k.start()
        copy_v.start()
        copy_k.wait_send()
        copy_v.wait_send()


def _attn_body(x_ref, wq_ref, k_ref, v_ref, wo_ref, out_ref):
    qt = pl.program_id(0)
    h = pl.program_id(1)

    xm = x_ref[0]
    q = jnp.dot(xm, wq_ref[...], preferred_element_type=jnp.float32)
    k = k_ref[0, :, 0, :]
    s = lax.dot_general(
        q, k, (((1,), (1,)), ((), ())), preferred_element_type=jnp.float32
    )
    s = s * SCALE
    row = lax.broadcasted_iota(jnp.int32, s.shape, 0) + qt * QT
    col = lax.broadcasted_iota(jnp.int32, s.shape, 1)
    s = jnp.where((col // BLK) <= (row // BLK), s, -1e9)
    m = jnp.max(s, axis=1, keepdims=True)
    w = jnp.exp(s - m)
    w = w / jnp.sum(w, axis=1, keepdims=True)
    ctx = jnp.dot(w, v_ref[0, :, 0, :], preferred_element_type=jnp.float32)
    contrib = jnp.dot(ctx, wo_ref[...], preferred_element_type=jnp.float32)

    @pl.when(h == 0)
    def _():
        out_ref[...] = contrib[None]

    @pl.when(h > 0)
    def _():
        out_ref[...] = out_ref[...] + contrib[None]


def kernel(x, Wq, K_ext, V_ext, Wo):
    B, Sq, Dm = x.shape
    _, Skv, Hq, Dh = K_ext.shape

    kfull, vfull = pl.pallas_call(
        _bcast_body,
        out_shape=[
            jax.ShapeDtypeStruct((B, Skv, Hq, Dh), jnp.float32),
            jax.ShapeDtypeStruct((B, Skv, Hq, Dh), jnp.float32),
        ],
        in_specs=[
            pl.BlockSpec(memory_space=pltpu.VMEM),
            pl.BlockSpec(memory_space=pltpu.VMEM),
        ],
        out_specs=[
            pl.BlockSpec(memory_space=pltpu.VMEM),
            pl.BlockSpec(memory_space=pltpu.VMEM),
        ],
        scratch_shapes=[
            pltpu.SemaphoreType.DMA((2,)),
            pltpu.SemaphoreType.DMA((2,)),
        ],
        compiler_params=pltpu.CompilerParams(collective_id=0),
    )(K_ext, V_ext)

    n_qt = Sq // QT
    out = pl.pallas_call(
        _attn_body,
        grid=(n_qt, Hq),
        in_specs=[
            pl.BlockSpec((1, QT, Dm), lambda qt, h: (0, qt, 0)),
            pl.BlockSpec((Dm, Dh), lambda qt, h: (0, h)),
            pl.BlockSpec((1, Skv, 1, Dh), lambda qt, h: (0, 0, h, 0)),
            pl.BlockSpec((1, Skv, 1, Dh), lambda qt, h: (0, 0, h, 0)),
            pl.BlockSpec((Dh, Dm), lambda qt, h: (h, 0)),
        ],
        out_specs=pl.BlockSpec((1, QT, Dm), lambda qt, h: (0, qt, 0)),
        out_shape=jax.ShapeDtypeStruct((B, Sq, Dm), jnp.float32),
        compiler_params=pltpu.CompilerParams(
            dimension_semantics=("arbitrary", "arbitrary"),
        ),
    )(x, Wq, kfull, vfull, Wo)
    return out


# baseline (device time: 895374 ns/iter reference)
import jax
import jax.numpy as jnp
from jax import lax
from jax.experimental import pallas as pl
from jax.experimental.pallas import tpu as pltpu

N_DEV = 8
SCALE = 0.08838834764831843
BLK = 64
QT = 512


def _bcast_body(k_ref, v_ref, kout_ref, vout_ref, send_sems, recv_sems):
    my = lax.axis_index("i")
    right = lax.rem(my + 1, N_DEV)

    copy_k = pltpu.make_async_remote_copy(
        src_ref=kout_ref,
        dst_ref=kout_ref,
        send_sem=send_sems.at[0],
        recv_sem=recv_sems.at[0],
        device_id=(right,),
        device_id_type=pl.DeviceIdType.MESH,
    )
    copy_v = pltpu.make_async_remote_copy(
        src_ref=vout_ref,
        dst_ref=vout_ref,
        send_sem=send_sems.at[1],
        recv_sem=recv_sems.at[1],
        device_id=(right,),
        device_id_type=pl.DeviceIdType.MESH,
    )

    @pl.when(my == 0)
    def _():
        kout_ref[...] = k_ref[...]
        vout_ref[...] = v_ref[...]

    @pl.when(my > 0)
    def _():
        copy_k.wait_recv()
        copy_v.wait_recv()

    @pl.when(my < N_DEV - 1)
    def _():
        copy_k.start()
        copy_v.start()
        copy_k.wait_send()
        copy_v.wait_send()


def _attn_body(x_ref, wq_ref, k_ref, v_ref, wo_ref, out_ref):
    qt = pl.program_id(0)
    h = pl.program_id(1)

    xm = x_ref[0]
    q = jnp.dot(xm, wq_ref[...], preferred_element_type=jnp.float32)
    k = k_ref[...]
    s = lax.dot_general(
        q, k, (((1,), (1,)), ((), ())), preferred_element_type=jnp.float32
    )
    s = s * SCALE
    row = lax.broadcasted_iota(jnp.int32, s.shape, 0) + qt * QT
    col = lax.broadcasted_iota(jnp.int32, s.shape, 1)
    s = jnp.where((col // BLK) <= (row // BLK), s, -1e9)
    m = jnp.max(s, axis=1, keepdims=True)
    w = jnp.exp(s - m)
    w = w / jnp.sum(w, axis=1, keepdims=True)
    ctx = jnp.dot(w, v_ref[...], preferred_element_type=jnp.float32)
    contrib = jnp.dot(ctx, wo_ref[...], preferred_element_type=jnp.float32)

    @pl.when(h == 0)
    def _():
        out_ref[...] = contrib[None]

    @pl.when(h > 0)
    def _():
        out_ref[...] = out_ref[...] + contrib[None]


def kernel(x, Wq, K_ext, V_ext, Wo):
    B, Sq, Dm = x.shape
    _, Skv, Hq, Dh = K_ext.shape

    k2 = K_ext.reshape(Skv, Hq * Dh)
    v2 = V_ext.reshape(Skv, Hq * Dh)

    kfull, vfull = pl.pallas_call(
        _bcast_body,
        out_shape=[
            jax.ShapeDtypeStruct((Skv, Hq * Dh), jnp.float32),
            jax.ShapeDtypeStruct((Skv, Hq * Dh), jnp.float32),
        ],
        in_specs=[
            pl.BlockSpec(memory_space=pltpu.VMEM),
            pl.BlockSpec(memory_space=pltpu.VMEM),
        ],
        out_specs=[
            pl.BlockSpec(memory_space=pltpu.VMEM),
            pl.BlockSpec(memory_space=pltpu.VMEM),
        ],
        scratch_shapes=[
            pltpu.SemaphoreType.DMA((2,)),
            pltpu.SemaphoreType.DMA((2,)),
        ],
    )(k2, v2)

    n_qt = Sq // QT
    out = pl.pallas_call(
        _attn_body,
        grid=(n_qt, Hq),
        in_specs=[
            pl.BlockSpec((1, QT, Dm), lambda qt, h: (0, qt, 0)),
            pl.BlockSpec((Dm, Dh), lambda qt, h: (0, h)),
            pl.BlockSpec((Skv, Dh), lambda qt, h: (0, h)),
            pl.BlockSpec((Skv, Dh), lambda qt, h: (0, h)),
            pl.BlockSpec((Dh, Dm), lambda qt, h: (h, 0)),
        ],
        out_specs=pl.BlockSpec((1, QT, Dm), lambda qt, h: (0, qt, 0)),
        out_shape=jax.ShapeDtypeStruct((B, Sq, Dm), jnp.float32),
        compiler_params=pltpu.CompilerParams(
            dimension_semantics=("arbitrary", "arbitrary"),
        ),
    )(x, Wq, kfull, vfull, Wo)
    return out


# device time: 388417 ns/iter; 2.3052x vs baseline; 2.3052x over previous
import jax
import jax.numpy as jnp
from jax import lax
from jax.experimental import pallas as pl
from jax.experimental.pallas import tpu as pltpu

N_DEV = 8
SCALE = 0.08838834764831843
BLK = 64
QT = 512


NC = 8
CH = 256


def _bcast_body(k_ref, v_ref, kout_ref, vout_ref, ks_sems, kr_sems, vs_sems, vr_sems):
    my = lax.axis_index("i")
    right = lax.rem(my + 1, N_DEV)

    def mk(src, dst, ssem, rsem, c):
        return pltpu.make_async_remote_copy(
            src_ref=src.at[pl.ds(c * CH, CH), :],
            dst_ref=dst.at[pl.ds(c * CH, CH), :],
            send_sem=ssem.at[c],
            recv_sem=rsem.at[c],
            device_id=(right,),
            device_id_type=pl.DeviceIdType.MESH,
        )

    @pl.when(my == 0)
    def _():
        for c in range(NC):
            mk(k_ref, kout_ref, ks_sems, kr_sems, c).start()
            mk(v_ref, vout_ref, vs_sems, vr_sems, c).start()
        kout_ref[...] = k_ref[...]
        vout_ref[...] = v_ref[...]
        for c in range(NC):
            mk(k_ref, kout_ref, ks_sems, kr_sems, c).wait_send()
            mk(v_ref, vout_ref, vs_sems, vr_sems, c).wait_send()

    @pl.when((my > 0) & (my < N_DEV - 1))
    def _():
        for c in range(NC):
            kc = mk(kout_ref, kout_ref, ks_sems, kr_sems, c)
            vc = mk(vout_ref, vout_ref, vs_sems, vr_sems, c)
            kc.wait_recv()
            kc.start()
            vc.wait_recv()
            vc.start()
        for c in range(NC):
            mk(kout_ref, kout_ref, ks_sems, kr_sems, c).wait_send()
            mk(vout_ref, vout_ref, vs_sems, vr_sems, c).wait_send()

    @pl.when(my == N_DEV - 1)
    def _():
        for c in range(NC):
            mk(kout_ref, kout_ref, ks_sems, kr_sems, c).wait_recv()
            mk(vout_ref, vout_ref, vs_sems, vr_sems, c).wait_recv()


def _attn_body(x_ref, wq_ref, k_ref, v_ref, wo_ref, out_ref):
    qt = pl.program_id(0)
    h = pl.program_id(1)

    xm = x_ref[0]
    q = jnp.dot(xm, wq_ref[...], preferred_element_type=jnp.float32)
    k = k_ref[...]
    s = lax.dot_general(
        q, k, (((1,), (1,)), ((), ())), preferred_element_type=jnp.float32
    )
    s = s * SCALE
    row = lax.broadcasted_iota(jnp.int32, s.shape, 0) + qt * QT
    col = lax.broadcasted_iota(jnp.int32, s.shape, 1)
    s = jnp.where((col // BLK) <= (row // BLK), s, -1e9)
    m = jnp.max(s, axis=1, keepdims=True)
    w = jnp.exp(s - m)
    w = w / jnp.sum(w, axis=1, keepdims=True)
    ctx = jnp.dot(w, v_ref[...], preferred_element_type=jnp.float32)
    contrib = jnp.dot(ctx, wo_ref[...], preferred_element_type=jnp.float32)

    @pl.when(h == 0)
    def _():
        out_ref[...] = contrib[None]

    @pl.when(h > 0)
    def _():
        out_ref[...] = out_ref[...] + contrib[None]


def kernel(x, Wq, K_ext, V_ext, Wo):
    B, Sq, Dm = x.shape
    _, Skv, Hq, Dh = K_ext.shape

    k2 = K_ext.reshape(Skv, Hq * Dh)
    v2 = V_ext.reshape(Skv, Hq * Dh)

    kfull, vfull = pl.pallas_call(
        _bcast_body,
        out_shape=[
            jax.ShapeDtypeStruct((Skv, Hq * Dh), jnp.float32),
            jax.ShapeDtypeStruct((Skv, Hq * Dh), jnp.float32),
        ],
        in_specs=[
            pl.BlockSpec(memory_space=pltpu.VMEM),
            pl.BlockSpec(memory_space=pltpu.VMEM),
        ],
        out_specs=[
            pl.BlockSpec(memory_space=pltpu.VMEM),
            pl.BlockSpec(memory_space=pltpu.VMEM),
        ],
        scratch_shapes=[
            pltpu.SemaphoreType.DMA((NC,)),
            pltpu.SemaphoreType.DMA((NC,)),
            pltpu.SemaphoreType.DMA((NC,)),
            pltpu.SemaphoreType.DMA((NC,)),
        ],
    )(k2, v2)

    n_qt = Sq // QT
    out = pl.pallas_call(
        _attn_body,
        grid=(n_qt, Hq),
        in_specs=[
            pl.BlockSpec((1, QT, Dm), lambda qt, h: (0, qt, 0)),
            pl.BlockSpec((Dm, Dh), lambda qt, h: (0, h)),
            pl.BlockSpec((Skv, Dh), lambda qt, h: (0, h)),
            pl.BlockSpec((Skv, Dh), lambda qt, h: (0, h)),
            pl.BlockSpec((Dh, Dm), lambda qt, h: (h, 0)),
        ],
        out_specs=pl.BlockSpec((1, QT, Dm), lambda qt, h: (0, qt, 0)),
        out_shape=jax.ShapeDtypeStruct((B, Sq, Dm), jnp.float32),
        compiler_params=pltpu.CompilerParams(
            dimension_semantics=("arbitrary", "arbitrary"),
        ),
    )(x, Wq, kfull, vfull, Wo)
    return out
